# Initial kernel scaffold; baseline (speedup 1.0000x reference)
import jax
import jax.numpy as jnp
from jax.experimental import pallas as pl


def kernel(x, W_emb, b_emb, Ws1, bs1, Ws2, bs2, Wa1, ba1, Wa2, ba2, Wo1, bo1, Wo2, bo2):
    raise NotImplementedError("write your pallas kernel here")


# speedup vs baseline: 4.1017x; 4.1017x over previous
import jax
import jax.numpy as jnp
from jax.experimental import pallas as pl
from jax.experimental.pallas import tpu as pltpu

TOKENS = 2048
INPUT = 1024
EMB = 512
NB = 8
NJ = 4
NCL = 128
TILE = 256


def _addr_flat(logits):
    a0 = (logits[:, 1:2] > logits[:, 0:1]).astype(jnp.int32)
    a1 = (logits[:, 3:4] > logits[:, 2:3]).astype(jnp.int32)
    a2 = (logits[:, 5:6] > logits[:, 4:5]).astype(jnp.int32)
    return 4 * a0 + 2 * a1 + a2


def _moe(h, W_ref, b_ref, flat, relu_out):
    acc = None
    for e in range(NB):
        z = jnp.dot(h, W_ref[e], preferred_element_type=jnp.float32) + b_ref[e]
        if relu_out:
            z = jnp.maximum(z, 0.0)
        sel = jnp.where(flat == e, z, 0.0)
        acc = sel if acc is None else acc + sel
    return acc


def _body(x_ref, W_emb_ref, b_emb_ref, Ws1_ref, bs1_ref, Ws2_ref, bs2_ref,
          Wa1_ref, ba1_ref, Wa2_ref, ba2_ref, Wo1_ref, bo1_ref, Wo2_ref,
          bo2_ref, out_ref):
    x = x_ref[...]
    state = jnp.dot(x, W_emb_ref[...], preferred_element_type=jnp.float32) + b_emb_ref[...]
    initial = state

    h0 = jnp.maximum(jnp.dot(state, Wa1_ref[0], preferred_element_type=jnp.float32) + ba1_ref[0], 0.0)
    logits = jnp.dot(h0, Wa2_ref[0], preferred_element_type=jnp.float32) + ba2_ref[0]
    flat = _addr_flat(logits)

    for i in range(NJ + 1):
        h1 = _moe(state, Ws1_ref, bs1_ref, flat, True)
        t2 = jnp.maximum(_moe(h1, Ws2_ref, bs2_ref, flat, False), 0.0)
        norm = jnp.sqrt(jnp.sum(state * state, axis=1, keepdims=True))
        normalized = t2 / (norm + 1e-6)
        if i == NJ:
            final = normalized + initial
            break
        g1 = _moe(normalized, Wa1_ref, ba1_ref, flat, True)
        logits = _moe(g1, Wa2_ref, ba2_ref, flat, False)
        flat = _addr_flat(logits)
        rw = i / max(1, NJ - 1)
        state = normalized + rw * initial

    h = jnp.maximum(jnp.dot(final, Wo1_ref[...], preferred_element_type=jnp.float32) + bo1_ref[...], 0.0)
    out_ref[...] = jnp.dot(h, Wo2_ref[...], preferred_element_type=jnp.float32) + bo2_ref[...]


def kernel(x, W_emb, b_emb, Ws1, bs1, Ws2, bs2, Wa1, ba1, Wa2, ba2, Wo1, bo1, Wo2, bo2):
    Wa2p = jnp.pad(Wa2, ((0, 0), (0, 0), (0, NCL - Wa2.shape[-1])))
    ba2p = jnp.pad(ba2, ((0, 0), (0, NCL - ba2.shape[-1])))

    full = lambda r: pl.BlockSpec(None, lambda i: (0,) * r)
    grid = TOKENS // TILE
    out = pl.pallas_call(
        _body,
        grid=(grid,),
        in_specs=[
            pl.BlockSpec((TILE, INPUT), lambda i: (i, 0)),
            full(2), full(1), full(3), full(2), full(3), full(2),
            full(3), full(2), full(3), full(2), full(2), full(1),
            full(2), full(1),
        ],
        out_specs=pl.BlockSpec((TILE, NCL), lambda i: (i, 0)),
        out_shape=jax.ShapeDtypeStruct((TOKENS, NCL), jnp.float32),
    )(x, W_emb, b_emb, Ws1, bs1, Ws2, bs2, Wa1, ba1, Wa2p, ba2p, Wo1, bo1, Wo2, bo2)
    return out
